# Initial kernel scaffold; baseline (speedup 1.0000x reference)
import functools

import jax
import jax.numpy as jnp
from jax import lax
from jax.experimental import pallas as pl
from jax.experimental.pallas import tpu as pltpu
from jax.experimental.pallas import tpu_sc as plsc

_B = 16
_V = 50000
_F = 100000
_NC = 2
_NS = 16
_K = 800
_NCHUNK = 8
_FP = _NS * _NCHUNK * _K
_VOUT = 50048
_DRN = _VOUT // _NS
_VP = _NS * 4 * _K
_PAD_ROW = _VOUT
_RB = 3128
_W = 25


def _sc_body(vstk, f0, f1, f2, out, idxa, idxb, idxc, t0, t1, acc,
             semg, sems0, sems1):
    cid = lax.axis_index("c")
    sid = lax.axis_index("s")

    zeros16 = jnp.zeros((16,), jnp.float32)

    def zrow(i, carry):
        t0[i, pl.ds(0, 16)] = zeros16
        t0[i, pl.ds(9, 16)] = zeros16
        return carry

    lax.fori_loop(0, _K, zrow, 0)
    for z in range(4):
        pltpu.sync_copy(t0, acc.at[pl.ds((sid * 4 + z) * _K, _K)])
    plsc.subcore_barrier()

    pltpu.sync_copy(f0.at[sid], idxa)
    pltpu.sync_copy(f1.at[sid], idxb)
    pltpu.sync_copy(f2.at[sid], idxc)

    vhalf = vstk.at[cid]
    bufs = (t0, t1)
    ssems = (sems0, sems1)
    pend = [None, None]
    for j in range(_NCHUNK):
        buf = bufs[j % 2]
        sem = ssems[j % 2]
        if pend[j % 2] is not None:
            for d in pend[j % 2]:
                d.wait()
        pltpu.async_copy(vhalf.at[idxa.at[j]], buf, semg).wait()
        pltpu.async_copy(vhalf.at[idxb.at[j]], buf, semg, add=True).wait()
        pltpu.async_copy(vhalf.at[idxc.at[j]], buf, semg, add=True).wait()
        pend[j % 2] = tuple(
            pltpu.async_copy(buf, acc.at[idx.at[j]], sem, add=True)
            for idx in (idxa, idxb, idxc))
    for p in pend:
        for d in p:
            d.wait()
    plsc.subcore_barrier()
    pltpu.sync_copy(acc.at[pl.ds(sid * _DRN, _DRN)],
                    out.at[cid, pl.ds(sid * _DRN, _DRN)])


_sc_scatter = functools.partial(
    pl.kernel,
    out_type=jax.ShapeDtypeStruct((_NC, _VOUT, _W), jnp.float32),
    mesh=plsc.VectorSubcoreMesh(
        core_axis_name="c", subcore_axis_name="s",
        num_cores=_NC, num_subcores=_NS),
    scratch_types=[
        pltpu.VMEM((_NCHUNK, _K), jnp.int32),
        pltpu.VMEM((_NCHUNK, _K), jnp.int32),
        pltpu.VMEM((_NCHUNK, _K), jnp.int32),
        pltpu.VMEM((_K, _W), jnp.float32),
        pltpu.VMEM((_K, _W), jnp.float32),
        pltpu.VMEM_SHARED((_VP, _W), jnp.float32),
        pltpu.SemaphoreType.DMA,
        pltpu.SemaphoreType.DMA,
        pltpu.SemaphoreType.DMA,
    ],
)(_sc_body)


def _tc_body(s_ref, v_ref, o_ref):
    h = pl.program_id(0)
    i = pl.program_id(1)
    s = s_ref[0]
    v = v_ref[0]
    occ = s[:, 24:25] * (1.0 / 3.0)
    deg = jnp.maximum(occ * 2.0, 1.0)
    nsq = jnp.zeros((_RB, 8), jnp.float32)
    for c in range(3):
        sc = s[:, c * 8:(c + 1) * 8]
        vc = v[:, c * 8:(c + 1) * 8]
        lap = (sc - occ * vc) / deg - vc
        nsq = nsq + lap * lap
    part = jnp.sum(jnp.sqrt(nsq))

    @pl.when(jnp.logical_and(h == 0, i == 0))
    def _():
        o_ref[0, 0] = jnp.float32(0.0)

    o_ref[0, 0] += part


def kernel(vert1, face):
    nb, nv, _ = vert1.shape
    nf = face.shape[0]
    vt = jnp.transpose(vert1.reshape(2, nb // 2, nv, 3), (0, 2, 3, 1))
    vt = vt.reshape(2, nv, 24)
    vstk = jnp.concatenate([vt, jnp.ones((2, nv, 1), jnp.float32)], axis=2)
    vstk = jnp.pad(vstk, ((0, 0), (0, _VP - nv), (0, 0)))

    fi = face.astype(jnp.int32)
    pad = _PAD_ROW + (jnp.arange(_FP - nf, dtype=jnp.int32) % 64)
    f0 = jnp.concatenate([fi[:, 0], pad]).reshape(_NS, _NCHUNK, _K)
    f1 = jnp.concatenate([fi[:, 1], pad]).reshape(_NS, _NCHUNK, _K)
    f2 = jnp.concatenate([fi[:, 2], pad]).reshape(_NS, _NCHUNK, _K)

    s_arr = _sc_scatter(vstk, f0, f1, f2)

    tot = pl.pallas_call(
        _tc_body,
        grid=(2, _VOUT // _RB),
        in_specs=[
            pl.BlockSpec((1, _RB, _W), lambda h, i: (h, i, 0)),
            pl.BlockSpec((1, _RB, _W), lambda h, i: (h, i, 0)),
        ],
        out_specs=pl.BlockSpec((1, 1), lambda h, i: (0, 0)),
        out_shape=jax.ShapeDtypeStruct((1, 1), jnp.float32),
    )(s_arr, vstk)
    return tot[0, 0] * (1.0 / (nb * nv))


# speedup vs baseline: 175.0214x; 175.0214x over previous
import functools

import jax
import jax.numpy as jnp
from jax import lax
from jax.experimental import pallas as pl
from jax.experimental.pallas import tpu as pltpu
from jax.experimental.pallas import tpu_sc as plsc

_B = 16
_V = 50000
_F = 100000
_NC = 2
_NS = 16
_K = 160
_NCHUNK = 40
_CPT = _NCHUNK * _K
_FP = _NS * _CPT
_VOUT = 50048
_DRN = _VOUT // _NS
_VP = 50176
_ZPT = _VP // _NS
_PAD_ROW = _VOUT
_RB = 3128
_W = 32


def _sc_body(verta, vertb, f0, f1, f2, zrows, out,
             ia0, ia1, ia2, ib0, ib1, ib2, ic0, ic1, ic2,
             ts0, ts1, gb, gc, acc, semg, semi, sems0, sems1):
    cid = lax.axis_index("c")
    sid = lax.axis_index("s")

    nfull = _ZPT // _K
    for z in range(nfull):
        pltpu.sync_copy(zrows, acc.at[pl.ds(sid * _ZPT + z * _K, _K)])
    rem = _ZPT - nfull * _K
    if rem:
        pltpu.sync_copy(zrows.at[pl.ds(0, rem)],
                        acc.at[pl.ds(sid * _ZPT + nfull * _K, rem)])
    plsc.subcore_barrier()

    ias = (ia0, ia1, ia2)
    ibs = (ib0, ib1, ib2)
    ics = (ic0, ic1, ic2)

    def load_idx(j):
        s = j % 3
        base = sid * _CPT + j * _K
        return (
            pltpu.async_copy(f0.at[pl.ds(base, _K)], ias[s], semi),
            pltpu.async_copy(f1.at[pl.ds(base, _K)], ibs[s], semi),
            pltpu.async_copy(f2.at[pl.ds(base, _K)], ics[s], semi),
        )

    def addrows(ts):
        def body(i, carry):
            for cc in range(2):
                sl = pl.ds(cc * 16, 16)
                ts[i, sl] = ts[i, sl] + gb[i, sl] + gc[i, sl]
            return carry

        lax.fori_loop(0, _K, body, 0)

    def main(vh):
        bufs = (ts0, ts1)
        ssems = (sems0, sems1)
        pend = [None, None]
        ipend = [None, None, None]
        ipend[0] = load_idx(0)
        for j in range(_NCHUNK):
            s = j % 3
            b = j % 2
            ts = bufs[b]
            sem = ssems[b]
            if pend[b] is not None:
                for d in pend[b]:
                    d.wait()
            if j + 1 < _NCHUNK:
                ipend[(j + 1) % 3] = load_idx(j + 1)
            for d in ipend[s]:
                d.wait()
            ia, ib, ic = ias[s], ibs[s], ics[s]
            da = pltpu.async_copy(vh.at[ia], ts, semg)
            db = pltpu.async_copy(vh.at[ib], gb, semg)
            dc = pltpu.async_copy(vh.at[ic], gc, semg)
            da.wait()
            db.wait()
            dc.wait()
            addrows(ts)
            pend[b] = tuple(
                pltpu.async_copy(ts, acc.at[ix], sem, add=True)
                for ix in (ia, ib, ic))
        for p in pend:
            for d in (p or ()):
                d.wait()

    @pl.when(cid == 0)
    def _():
        main(verta)

    @pl.when(cid == 1)
    def _():
        main(vertb)

    plsc.subcore_barrier()
    pltpu.sync_copy(acc.at[pl.ds(sid * _DRN, _DRN)],
                    out.at[cid, pl.ds(sid * _DRN, _DRN)])


_sc_scatter = functools.partial(
    pl.kernel,
    out_type=pltpu.HBM((_NC, _VOUT, _W), jnp.float32),
    mesh=plsc.VectorSubcoreMesh(
        core_axis_name="c", subcore_axis_name="s",
        num_cores=_NC, num_subcores=_NS),
    scratch_types=(
        [pltpu.VMEM((_K,), jnp.int32)] * 9
        + [pltpu.VMEM((_K, _W), jnp.float32)] * 4
        + [
            pltpu.VMEM_SHARED((_VP, _W), jnp.float32),
            pltpu.SemaphoreType.DMA,
            pltpu.SemaphoreType.DMA,
            pltpu.SemaphoreType.DMA,
            pltpu.SemaphoreType.DMA,
        ]
    ),
    compiler_params=pltpu.CompilerParams(use_tc_tiling_on_sc=False),
)(_sc_body)


def _tc_body(s_ref, va_ref, vb_ref, o_ref):
    h = pl.program_id(0)
    i = pl.program_id(1)
    s = s_ref[0]
    v = jnp.where(h == 0, va_ref[...], vb_ref[...])
    occ = s[:, 24:25] * (1.0 / 3.0)
    deg = jnp.maximum(occ * 2.0, 1.0)
    nsq = jnp.zeros((_RB, 8), jnp.float32)
    for c in range(3):
        sc = s[:, c * 8:(c + 1) * 8]
        vc = v[:, c * 8:(c + 1) * 8]
        lap = (sc - occ * vc) / deg - vc
        nsq = nsq + lap * lap
    part = jnp.sum(jnp.sqrt(nsq))

    @pl.when(jnp.logical_and(h == 0, i == 0))
    def _():
        o_ref[...] = jnp.zeros((1, 1), jnp.float32)

    o_ref[...] += part


def kernel(vert1, face):
    nb, nv, _ = vert1.shape
    nf = face.shape[0]
    vt = jnp.transpose(vert1.reshape(2, nb // 2, nv, 3), (0, 2, 3, 1))
    vt = vt.reshape(2, nv, 24)
    vstk = jnp.concatenate([vt, jnp.ones((2, nv, 1), jnp.float32)], axis=2)
    vstk = jnp.pad(vstk, ((0, 0), (0, _VP - nv), (0, _W - 25)))
    verta, vertb = vstk[0], vstk[1]

    fi = face.astype(jnp.int32)
    pad = _PAD_ROW + (jnp.arange(_FP - nf, dtype=jnp.int32) % 64)
    f0 = jnp.concatenate([fi[:, 0], pad])
    f1 = jnp.concatenate([fi[:, 1], pad])
    f2 = jnp.concatenate([fi[:, 2], pad])

    s_arr = _sc_scatter(verta, vertb, f0, f1, f2,
                        jnp.zeros((_K, _W), jnp.float32))

    tot = pl.pallas_call(
        _tc_body,
        grid=(2, _VOUT // _RB),
        in_specs=[
            pl.BlockSpec((1, _RB, _W), lambda h, i: (h, i, 0)),
            pl.BlockSpec((_RB, _W), lambda h, i: (i, 0)),
            pl.BlockSpec((_RB, _W), lambda h, i: (i, 0)),
        ],
        out_specs=pl.BlockSpec((1, 1), lambda h, i: (0, 0)),
        out_shape=jax.ShapeDtypeStruct((1, 1), jnp.float32),
    )(s_arr, verta, vertb)
    return tot[0, 0] * (1.0 / (nb * nv))


# speedup vs baseline: 179.0050x; 1.0228x over previous
import functools

import jax
import jax.numpy as jnp
from jax import lax
from jax.experimental import pallas as pl
from jax.experimental.pallas import tpu as pltpu
from jax.experimental.pallas import tpu_sc as plsc

_B = 16
_V = 50000
_F = 100000
_NC = 2
_NS = 16
_K = 160
_NCHUNK = 40
_CPT = _NCHUNK * _K
_FP = _NS * _CPT
_VOUT = 50048
_DRN = _VOUT // _NS
_VP = 50176
_ZPT = _VP // _NS
_PAD_ROW = _VOUT
_RB = 3128
_W = 32


def _sc_body(verta, vertb, f0, f1, f2, zrows, out,
             ia0, ia1, ia2, ib0, ib1, ib2, ic0, ic1, ic2,
             ts0, ts1, gb, gc, acc, semg, semi, sems0, sems1):
    cid = lax.axis_index("c")
    sid = lax.axis_index("s")

    nfull = _ZPT // _K
    for z in range(nfull):
        pltpu.sync_copy(zrows, acc.at[pl.ds(sid * _ZPT + z * _K, _K)])
    rem = _ZPT - nfull * _K
    if rem:
        pltpu.sync_copy(zrows.at[pl.ds(0, rem)],
                        acc.at[pl.ds(sid * _ZPT + nfull * _K, rem)])
    plsc.subcore_barrier()

    ias = (ia0, ia1, ia2)
    ibs = (ib0, ib1, ib2)
    ics = (ic0, ic1, ic2)

    def load_idx(j):
        s = j % 3
        base = sid * _CPT + j * _K
        return (
            pltpu.async_copy(f0.at[pl.ds(base, _K)], ias[s], semi),
            pltpu.async_copy(f1.at[pl.ds(base, _K)], ibs[s], semi),
            pltpu.async_copy(f2.at[pl.ds(base, _K)], ics[s], semi),
        )

    def addrows(ts):
        def body(i, carry):
            for cc in range(2):
                sl = pl.ds(cc * 16, 16)
                ts[i, sl] = ts[i, sl] + gb[i, sl] + gc[i, sl]
            return carry

        lax.fori_loop(0, _K, body, 0)

    def main(vh):
        bufs = (ts0, ts1)
        ssems = (sems0, sems1)
        pend = [None, None]
        ipend = [None, None, None]
        ipend[0] = load_idx(0)
        for j in range(_NCHUNK):
            s = j % 3
            b = j % 2
            ts = bufs[b]
            sem = ssems[b]
            if pend[b] is not None:
                for d in pend[b]:
                    d.wait()
            if j + 1 < _NCHUNK:
                ipend[(j + 1) % 3] = load_idx(j + 1)
            for d in ipend[s]:
                d.wait()
            ia, ib, ic = ias[s], ibs[s], ics[s]
            da = pltpu.async_copy(vh.at[ia], ts, semg)
            db = pltpu.async_copy(vh.at[ib], gb, semg)
            dc = pltpu.async_copy(vh.at[ic], gc, semg)
            da.wait()
            db.wait()
            dc.wait()
            addrows(ts)
            pend[b] = tuple(
                pltpu.async_copy(ts, acc.at[ix], sem, add=True)
                for ix in (ia, ib, ic))
        for p in pend:
            for d in (p or ()):
                d.wait()

    @pl.when(cid == 0)
    def _():
        main(verta)

    @pl.when(cid == 1)
    def _():
        main(vertb)

    plsc.subcore_barrier()
    pltpu.sync_copy(acc.at[pl.ds(sid * _DRN, _DRN)],
                    out.at[cid, pl.ds(sid * _DRN, _DRN)])


_sc_scatter = functools.partial(
    pl.kernel,
    out_type=pltpu.HBM((_NC, _VOUT, _W), jnp.float32),
    mesh=plsc.VectorSubcoreMesh(
        core_axis_name="c", subcore_axis_name="s",
        num_cores=_NC, num_subcores=_NS),
    scratch_types=(
        [pltpu.VMEM((_K,), jnp.int32)] * 9
        + [pltpu.VMEM((_K, _W), jnp.float32)] * 4
        + [
            pltpu.VMEM_SHARED((_VP, _W), jnp.float32),
            pltpu.SemaphoreType.DMA,
            pltpu.SemaphoreType.DMA,
            pltpu.SemaphoreType.DMA,
            pltpu.SemaphoreType.DMA,
        ]
    ),
    compiler_params=pltpu.CompilerParams(use_tc_tiling_on_sc=False),
)(_sc_body)


def _coord_select():
    m = [[0.0] * 8 for _ in range(_W)]
    for c in range(3):
        for b in range(8):
            m[c * 8 + b][b] = 1.0
    return jnp.asarray(m, dtype=jnp.float32)


def _tc_body(s_ref, va_ref, vb_ref, m_ref, o_ref):
    h = pl.program_id(0)
    i = pl.program_id(1)
    s = s_ref[0]
    v = jnp.where(h == 0, va_ref[...], vb_ref[...])
    occ = s[:, 24:25] * (1.0 / 3.0)
    deg = jnp.maximum(occ * 2.0, 1.0)
    lap = (s - occ * v) / deg - v
    nsq = jax.lax.dot_general(
        lap * lap, m_ref[...],
        (((1,), (0,)), ((), ())),
        preferred_element_type=jnp.float32)
    part = jnp.sum(jnp.sqrt(nsq))

    @pl.when(jnp.logical_and(h == 0, i == 0))
    def _():
        o_ref[...] = jnp.zeros((1, 1), jnp.float32)

    o_ref[...] += part


def kernel(vert1, face):
    nb, nv, _ = vert1.shape
    nf = face.shape[0]
    vt = jnp.transpose(vert1.reshape(2, nb // 2, nv, 3), (0, 2, 3, 1))
    vt = vt.reshape(2, nv, 24)
    vstk = jnp.concatenate([vt, jnp.ones((2, nv, 1), jnp.float32)], axis=2)
    vstk = jnp.pad(vstk, ((0, 0), (0, _VP - nv), (0, _W - 25)))
    verta, vertb = vstk[0], vstk[1]

    fi = face.astype(jnp.int32)
    pad = _PAD_ROW + (jnp.arange(_FP - nf, dtype=jnp.int32) % 64)
    f0 = jnp.concatenate([fi[:, 0], pad])
    f1 = jnp.concatenate([fi[:, 1], pad])
    f2 = jnp.concatenate([fi[:, 2], pad])

    s_arr = _sc_scatter(verta, vertb, f0, f1, f2,
                        jnp.zeros((_K, _W), jnp.float32))

    tot = pl.pallas_call(
        _tc_body,
        grid=(2, _VOUT // _RB),
        in_specs=[
            pl.BlockSpec((1, _RB, _W), lambda h, i: (h, i, 0)),
            pl.BlockSpec((_RB, _W), lambda h, i: (i, 0)),
            pl.BlockSpec((_RB, _W), lambda h, i: (i, 0)),
            pl.BlockSpec((_W, 8), lambda h, i: (0, 0)),
        ],
        out_specs=pl.BlockSpec((1, 1), lambda h, i: (0, 0)),
        out_shape=jax.ShapeDtypeStruct((1, 1), jnp.float32),
    )(s_arr, verta, vertb, _coord_select())
    return tot[0, 0] * (1.0 / (nb * nv))


# speedup vs baseline: 210.0603x; 1.1735x over previous
import functools

import jax
import jax.numpy as jnp
from jax import lax
from jax.experimental import pallas as pl
from jax.experimental.pallas import tpu as pltpu
from jax.experimental.pallas import tpu_sc as plsc

_B = 16
_V = 50000
_F = 100000
_NC = 2
_NS = 16
_K = 160
_NCHUNK = 40
_CPT = _NCHUNK * _K
_FP = _NS * _CPT
_VOUT = 50048
_DRN = _VOUT // _NS
_VP = 50176
_ZPT = _VP // _NS
_PAD_ROW = _VOUT
_RB = 3128
_W = 32


def _sc_body(verta, vertb, f0, f1, f2, zrows, out,
             ia0, ia1, ia2, ib0, ib1, ib2, ic0, ic1, ic2,
             ts0, ts1, gb, gc, acc, semg, semi, sems0, sems1):
    cid = lax.axis_index("c")
    sid = lax.axis_index("s")

    nfull = _ZPT // _K
    for z in range(nfull):
        pltpu.sync_copy(zrows, acc.at[pl.ds(sid * _ZPT + z * _K, _K)])
    rem = _ZPT - nfull * _K
    if rem:
        pltpu.sync_copy(zrows.at[pl.ds(0, rem)],
                        acc.at[pl.ds(sid * _ZPT + nfull * _K, rem)])
    plsc.subcore_barrier()

    ias = (ia0, ia1, ia2)
    ibs = (ib0, ib1, ib2)
    ics = (ic0, ic1, ic2)

    def load_idx(j):
        s = j % 3
        base = sid * _CPT + j * _K
        return (
            pltpu.async_copy(f0.at[pl.ds(base, _K)], ias[s], semi),
            pltpu.async_copy(f1.at[pl.ds(base, _K)], ibs[s], semi),
            pltpu.async_copy(f2.at[pl.ds(base, _K)], ics[s], semi),
        )

    def addrows(ts):
        def body(i, carry):
            for cc in range(2):
                sl = pl.ds(cc * 16, 16)
                ts[i, sl] = ts[i, sl] + gb[i, sl] + gc[i, sl]
            return carry

        lax.fori_loop(0, _K, body, 0)

    def main(vh):
        bufs = (ts0, ts1)
        ssems = (sems0, sems1)
        pend = [None, None]
        ipend = [None, None, None]
        ipend[0] = load_idx(0)
        for j in range(_NCHUNK):
            s = j % 3
            b = j % 2
            ts = bufs[b]
            sem = ssems[b]
            if pend[b] is not None:
                for d in pend[b]:
                    d.wait()
            if j + 1 < _NCHUNK:
                ipend[(j + 1) % 3] = load_idx(j + 1)
            for d in ipend[s]:
                d.wait()
            ia, ib, ic = ias[s], ibs[s], ics[s]
            da = pltpu.async_copy(vh.at[ia], ts, semg)
            db = pltpu.async_copy(vh.at[ib], gb, semg)
            dc = pltpu.async_copy(vh.at[ic], gc, semg)
            da.wait()
            db.wait()
            dc.wait()
            addrows(ts)
            pend[b] = tuple(
                pltpu.async_copy(ts, acc.at[ix], sem, add=True)
                for ix in (ia, ib, ic))
        for p in pend:
            for d in (p or ()):
                d.wait()

    @pl.when(cid == 0)
    def _():
        main(verta)

    @pl.when(cid == 1)
    def _():
        main(vertb)

    plsc.subcore_barrier()
    pltpu.sync_copy(acc.at[pl.ds(sid * _DRN, _DRN)],
                    out.at[cid, pl.ds(sid * _DRN, _DRN)])


_sc_scatter = functools.partial(
    pl.kernel,
    out_type=pltpu.HBM((_NC, _VOUT, _W), jnp.float32),
    mesh=plsc.VectorSubcoreMesh(
        core_axis_name="c", subcore_axis_name="s",
        num_cores=_NC, num_subcores=_NS),
    scratch_types=(
        [pltpu.VMEM((_K,), jnp.int32)] * 9
        + [pltpu.VMEM((_K, _W), jnp.float32)] * 4
        + [
            pltpu.VMEM_SHARED((_VP, _W), jnp.float32),
            pltpu.SemaphoreType.DMA,
            pltpu.SemaphoreType.DMA,
            pltpu.SemaphoreType.DMA,
            pltpu.SemaphoreType.DMA,
        ]
    ),
    compiler_params=pltpu.CompilerParams(use_tc_tiling_on_sc=False),
)(_sc_body)


def _coord_select():
    m = [[0.0] * 8 for _ in range(_W)]
    for c in range(3):
        for b in range(8):
            m[c * 8 + b][b] = 1.0
    return jnp.asarray(m, dtype=jnp.float32)


def _tc_body(s_ref, va_ref, vb_ref, m_ref, o_ref):
    h = pl.program_id(0)
    i = pl.program_id(1)
    s = s_ref[0]
    v = jnp.where(h == 0, va_ref[...], vb_ref[...])
    occ = s[:, 24:25] * (1.0 / 3.0)
    deg = jnp.maximum(occ * 2.0, 1.0)
    lap = (s - occ * v) / deg - v
    nsq = jax.lax.dot_general(
        lap * lap, m_ref[...],
        (((1,), (0,)), ((), ())),
        preferred_element_type=jnp.float32)
    part = jnp.sum(jnp.sqrt(nsq))

    @pl.when(jnp.logical_and(h == 0, i == 0))
    def _():
        o_ref[...] = jnp.zeros((1, 1), jnp.float32)

    o_ref[...] += part


def kernel(vert1, face):
    nb, nv, _ = vert1.shape
    nf = face.shape[0]
    def table(vh):
        vt = jnp.transpose(vh, (1, 2, 0)).reshape(nv, 24)
        vt = jnp.concatenate([vt, jnp.ones((nv, 1), jnp.float32)], axis=1)
        return jnp.pad(vt, ((0, _VP - nv), (0, _W - 25)))

    verta = table(vert1[:nb // 2])
    vertb = table(vert1[nb // 2:])

    fi = face.astype(jnp.int32)
    pad = _PAD_ROW + (jnp.arange(_FP - nf, dtype=jnp.int32) % 64)
    f0 = jnp.concatenate([fi[:, 0], pad])
    f1 = jnp.concatenate([fi[:, 1], pad])
    f2 = jnp.concatenate([fi[:, 2], pad])

    s_arr = _sc_scatter(verta, vertb, f0, f1, f2,
                        jnp.zeros((_K, _W), jnp.float32))

    tot = pl.pallas_call(
        _tc_body,
        grid=(2, _VOUT // _RB),
        in_specs=[
            pl.BlockSpec((1, _RB, _W), lambda h, i: (h, i, 0)),
            pl.BlockSpec((_RB, _W), lambda h, i: (i, 0)),
            pl.BlockSpec((_RB, _W), lambda h, i: (i, 0)),
            pl.BlockSpec((_W, 8), lambda h, i: (0, 0)),
        ],
        out_specs=pl.BlockSpec((1, 1), lambda h, i: (0, 0)),
        out_shape=jax.ShapeDtypeStruct((1, 1), jnp.float32),
    )(s_arr, verta, vertb, _coord_select())
    return tot[0, 0] * (1.0 / (nb * nv))
